# Initial kernel scaffold; baseline (speedup 1.0000x reference)
import jax
import jax.numpy as jnp
from jax.experimental import pallas as pl


def kernel(x, edge_index, W1_l, W1_r, b1, W2_l, W2_r, b2, W_pred, b_pred, W_diff, b_diff):
    raise NotImplementedError("write your pallas kernel here")


# speedup vs baseline: 4.6963x; 4.6963x over previous
import functools

import jax
import jax.numpy as jnp
from jax import lax
from jax.experimental import pallas as pl
from jax.experimental.pallas import tpu as pltpu
from jax.experimental.pallas import tpu_sc as plsc

NN = 10000
N_PAD = 10240
EE = 320000
DD = 128
DEXT = 144
NC = 2
NS = 16
CHUNK = 80
EDGES_PER_TILE = EE // (NC * NS)
N_CHUNKS = EDGES_PER_TILE // CHUNK
STRIPE = N_PAD // NS
ROW_BLK = 1024
N_BLKS = N_PAD // ROW_BLK


def _make_segsum(feat):
    mesh = plsc.VectorSubcoreMesh(core_axis_name="c", subcore_axis_name="s")

    @functools.partial(
        pl.kernel,
        mesh=mesh,
        compiler_params=pltpu.CompilerParams(use_tc_tiling_on_sc=False),
        out_type=jax.ShapeDtypeStruct((NC, N_PAD, feat), jnp.float32),
        scratch_types=[
            pltpu.VMEM_SHARED((N_PAD, feat), jnp.float32),
            pltpu.VMEM((CHUNK,), jnp.int32),
            pltpu.VMEM((CHUNK,), jnp.int32),
            pltpu.VMEM((CHUNK, feat), jnp.float32),
            pltpu.SemaphoreType.DMA,
        ],
    )
    def seg(table_hbm, src_hbm, dst_hbm, zeros_hbm, out_hbm,
            acc_sh, idx_s, idx_d, rows, sem):
        c = lax.axis_index("c")
        s = lax.axis_index("s")
        wid = c * NS + s
        pltpu.sync_copy(zeros_hbm, acc_sh.at[pl.ds(s * STRIPE, STRIPE)])
        plsc.subcore_barrier()
        base = wid * EDGES_PER_TILE

        @pl.loop(0, N_CHUNKS)
        def _(j):
            off = base + j * CHUNK
            pltpu.sync_copy(src_hbm.at[pl.ds(off, CHUNK)], idx_s)
            pltpu.sync_copy(dst_hbm.at[pl.ds(off, CHUNK)], idx_d)
            pltpu.async_copy(table_hbm.at[idx_s], rows, sem).wait()
            pltpu.sync_copy(rows, acc_sh.at[idx_d], add=True)

        plsc.subcore_barrier()
        pltpu.sync_copy(acc_sh.at[pl.ds(s * STRIPE, STRIPE)],
                        out_hbm.at[c].at[pl.ds(s * STRIPE, STRIPE)])

    return seg


_segsum_ext = _make_segsum(DEXT)
_segsum_d = _make_segsum(DD)


def _tc1_body(acc_ref, x_ref, wl_ref, wr_ref, b_ref, h_ref, inv_ref):
    ssum = acc_ref[0] + acc_ref[1]
    agg = ssum[:, :DD]
    deg = ssum[:, DD:DD + 1]
    inv = 1.0 / jnp.maximum(deg, 1.0)
    m = agg * inv
    h = (jnp.dot(m, wl_ref[...], preferred_element_type=jnp.float32)
         + jnp.dot(x_ref[...], wr_ref[...], preferred_element_type=jnp.float32)
         + b_ref[...])
    h_ref[...] = jnp.maximum(h, 0.0)
    inv_ref[...] = inv


def _tc2_body(acc_ref, h_ref, inv_ref, wl_ref, wr_ref, b_ref, whd_ref, bhd_ref,
              o1_ref, o2_ref):
    ssum = acc_ref[0] + acc_ref[1]
    m = ssum * inv_ref[...]
    h2 = (jnp.dot(m, wl_ref[...], preferred_element_type=jnp.float32)
          + jnp.dot(h_ref[...], wr_ref[...], preferred_element_type=jnp.float32)
          + b_ref[...])
    h2 = jnp.maximum(h2, 0.0)
    t = jnp.dot(h2, whd_ref[...], preferred_element_type=jnp.float32) + bhd_ref[...]
    preds = t[:, 0:1]
    diffs = jax.nn.sigmoid(t[:, 1:2])
    o1_ref[...] = preds - diffs
    o2_ref[...] = preds + diffs


def _full(shape):
    return pl.BlockSpec(shape, lambda j: tuple(0 for _ in shape))


def _tc_layer1(acc1, x_pad, W1_l, W1_r, b1):
    return pl.pallas_call(
        _tc1_body,
        grid=(N_BLKS,),
        in_specs=[
            pl.BlockSpec((NC, ROW_BLK, DEXT), lambda j: (0, j, 0)),
            pl.BlockSpec((ROW_BLK, DD), lambda j: (j, 0)),
            _full((DD, DD)),
            _full((DD, DD)),
            _full((1, DD)),
        ],
        out_specs=[
            pl.BlockSpec((ROW_BLK, DD), lambda j: (j, 0)),
            pl.BlockSpec((ROW_BLK, 1), lambda j: (j, 0)),
        ],
        out_shape=[
            jax.ShapeDtypeStruct((N_PAD, DD), jnp.float32),
            jax.ShapeDtypeStruct((N_PAD, 1), jnp.float32),
        ],
    )(acc1, x_pad, W1_l, W1_r, b1)


def _tc_layer2(acc2, h, inv, W2_l, W2_r, b2, W_hd, b_hd):
    return pl.pallas_call(
        _tc2_body,
        grid=(N_BLKS,),
        in_specs=[
            pl.BlockSpec((NC, ROW_BLK, DD), lambda j: (0, j, 0)),
            pl.BlockSpec((ROW_BLK, DD), lambda j: (j, 0)),
            pl.BlockSpec((ROW_BLK, 1), lambda j: (j, 0)),
            _full((DD, DD)),
            _full((DD, DD)),
            _full((1, DD)),
            _full((DD, DD)),
            _full((1, DD)),
        ],
        out_specs=[
            pl.BlockSpec((ROW_BLK, 1), lambda j: (j, 0)),
            pl.BlockSpec((ROW_BLK, 1), lambda j: (j, 0)),
        ],
        out_shape=[
            jax.ShapeDtypeStruct((N_PAD, 1), jnp.float32),
            jax.ShapeDtypeStruct((N_PAD, 1), jnp.float32),
        ],
    )(acc2, h, inv, W2_l, W2_r, b2, W_hd, b_hd)


def kernel(x, edge_index, W1_l, W1_r, b1, W2_l, W2_r, b2, W_pred, b_pred,
           W_diff, b_diff):
    src = edge_index[0]
    dst = edge_index[1]
    f32 = jnp.float32
    x_ext = jnp.concatenate(
        [x, jnp.ones((NN, 1), f32), jnp.zeros((NN, DEXT - DD - 1), f32)], axis=1)
    x_pad = jnp.concatenate([x, jnp.zeros((N_PAD - NN, DD), f32)], axis=0)
    zeros_ext = jnp.zeros((STRIPE, DEXT), f32)
    zeros_d = jnp.zeros((STRIPE, DD), f32)
    W_hd = jnp.concatenate(
        [W_pred, W_diff, jnp.zeros((DD, DD - 2), f32)], axis=1)
    b_hd = jnp.concatenate(
        [b_pred, b_diff, jnp.zeros((DD - 2,), f32)]).reshape(1, DD)

    acc1 = _segsum_ext(x_ext, src, dst, zeros_ext)
    h, inv = _tc_layer1(acc1, x_pad, W1_l, W1_r, b1.reshape(1, DD))
    acc2 = _segsum_d(h, src, dst, zeros_d)
    o1, o2 = _tc_layer2(acc2, h, inv, W2_l, W2_r, b2.reshape(1, DD), W_hd, b_hd)
    return (o1[:NN], o2[:NN])
